# baseline (speedup 1.0000x reference)
import functools

import jax
import jax.numpy as jnp
from jax import lax
from jax.experimental import pallas as pl
from jax.experimental.pallas import tpu as pltpu
from jax.experimental.pallas import tpu_sc as plsc

EMBED_DIM = 64
TBLOCK = 8192
HIST = 50
BATCH = 4096
LANES = 16


@functools.lru_cache(maxsize=None)
def _make_transpose(n_items):
    nblocks = (n_items + TBLOCK - 1) // TBLOCK
    n_rows_out = nblocks * TBLOCK

    def body(t_ref, out_ref, scratch):
        scratch[...] = t_ref[...].T
        out_ref[:, 0:EMBED_DIM] = scratch[pl.ds(0, TBLOCK // 2, 2), :]
        out_ref[:, EMBED_DIM : 2 * EMBED_DIM] = scratch[pl.ds(1, TBLOCK // 2, 2), :]

    return pl.pallas_call(
        body,
        grid=(nblocks,),
        in_specs=[pl.BlockSpec((EMBED_DIM, TBLOCK), lambda i: (0, i))],
        out_specs=pl.BlockSpec((TBLOCK // 2, 2 * EMBED_DIM), lambda i: (i, 0)),
        out_shape=jax.ShapeDtypeStruct((n_rows_out // 2, 2 * EMBED_DIM), jnp.float32),
        scratch_shapes=[pltpu.VMEM((TBLOCK, EMBED_DIM), jnp.float32)],
    )


@functools.lru_cache(maxsize=None)
def _make_gather(table_rows, n_workers):
    n_rows = BATCH * HIST
    b_per_w = n_rows // n_workers
    units = b_per_w // 128
    nbuf = 2
    jtile = EMBED_DIM // 8
    btile = BATCH // 128
    mesh = plsc.VectorSubcoreMesh(core_axis_name="c", subcore_axis_name="s")

    @functools.partial(
        pl.kernel,
        mesh=mesh,
        out_type=jax.ShapeDtypeStruct((HIST, jtile, btile, 8 * 128), jnp.float32),
        compiler_params=pltpu.CompilerParams(
            use_tc_tiling_on_sc=False, needs_layout_passes=False
        ),
        scratch_types=[
            pltpu.VMEM((b_per_w,), jnp.int32),
            pltpu.VMEM((128, EMBED_DIM), jnp.float32),
            pltpu.VMEM((128, EMBED_DIM), jnp.float32),
            pltpu.VMEM((EMBED_DIM * 128,), jnp.float32),
            pltpu.VMEM((EMBED_DIM * 128,), jnp.float32),
            pltpu.SemaphoreType.DMA,
            pltpu.SemaphoreType.DMA,
            pltpu.SemaphoreType.DMA,
            pltpu.SemaphoreType.DMA,
        ],
    )
    def k(table_hbm, idx_hbm, out_hbm, idx_v, buf0, buf1, tbuf0, tbuf1,
          gsem0, gsem1, osem0, osem1):
        wid = lax.axis_index("s") * 2 + lax.axis_index("c")
        base = wid * b_per_w
        pltpu.sync_copy(idx_hbm.at[pl.ds(base, b_per_w)], idx_v)
        bufs = (buf0, buf1)
        tbufs = (tbuf0, tbuf1)
        gsems = (gsem0, gsem1)
        osems = (osem0, osem1)
        iota16 = lax.iota(jnp.int32, LANES)
        ivs = [(g * LANES + iota16) * 128 for g in range(EMBED_DIM // LANES)]
        u0 = wid * units

        def start_gather(k_local):
            s = k_local % nbuf
            return pltpu.async_copy(
                table_hbm.at[idx_v.at[pl.ds(k_local * 128, 128)]],
                bufs[s],
                gsems[s],
            )

        def transpose_unit(s):
            buf, tbuf = bufs[s], tbufs[s]

            def body(r, _):
                for g in range(EMBED_DIM // LANES):
                    v = buf[r, pl.ds(g * LANES, LANES)]
                    plsc.store_scatter(tbuf, [ivs[g] + r], v)
                return 0

            lax.fori_loop(0, 128, body, 0)

        def start_out(k_local):
            s = k_local % nbuf
            u = u0 + k_local
            h = u // btile
            bt = u % btile
            return [
                pltpu.async_copy(
                    tbufs[s].at[pl.ds(jt * 1024, 1024)],
                    out_hbm.at[h, jt, bt, :],
                    osems[s],
                )
                for jt in range(jtile)
            ]

        def wait_out(copies):
            for c in copies:
                c.wait()

        gcopies = [None] * units
        ocopies = [None] * units
        for u in range(units):
            s = u % nbuf
            if u >= nbuf:
                wait_out(ocopies[u - nbuf])
            gcopies[u] = start_gather(u)
            if u >= 1:
                gcopies[u - 1].wait()
                transpose_unit((u - 1) % nbuf)
                ocopies[u - 1] = start_out(u - 1)
        gcopies[units - 1].wait()
        transpose_unit((units - 1) % nbuf)
        ocopies[units - 1] = start_out(units - 1)
        wait_out(ocopies[units - 2])
        wait_out(ocopies[units - 1])

    return k


def kernel(item_ids, item_emb):
    b, h = item_ids.shape
    n_items = item_emb.shape[0]
    ids = item_ids.T.reshape(b * h).astype(jnp.int32)
    packed = _make_transpose(n_items)(item_emb.T)
    table = packed.reshape(packed.shape[0] * 2, EMBED_DIM)
    out5 = _make_gather(table.shape[0], 32)(table, ids)
    out5 = out5.reshape(HIST, EMBED_DIM // 8, BATCH // 128, 8, 128)
    out = jnp.transpose(out5, (0, 1, 3, 2, 4)).reshape(HIST, EMBED_DIM, BATCH)
    return jnp.transpose(out, (2, 0, 1))
